# baseline (speedup 1.0000x reference)
import functools

import jax
import jax.numpy as jnp
from jax import lax
from jax.experimental import pallas as pl
from jax.experimental.pallas import tpu as pltpu
from jax.experimental.pallas import tpu_sc as plsc

_B = 16384
_D = 32
_NC = 2
_NS = 16
_LANES = 16
_NW = _NC * _NS
_BPW = _B // _NW
_GROUPS = _BPW // _LANES


def _make_sc_kernel():
    mesh = plsc.VectorSubcoreMesh(core_axis_name="c", subcore_axis_name="s")

    @functools.partial(
        pl.kernel,
        mesh=mesh,
        out_type=jax.ShapeDtypeStruct((_B,), jnp.float32),
        compiler_params=pltpu.CompilerParams(
            needs_layout_passes=False, use_tc_tiling_on_sc=False),
        scratch_types=[
            pltpu.VMEM((_BPW,), jnp.int32),
            pltpu.VMEM((_BPW,), jnp.int32),
            pltpu.VMEM((_BPW, _D), jnp.float32),
            pltpu.VMEM((_BPW, _D), jnp.float32),
            pltpu.VMEM((_BPW * _LANES + _LANES,), jnp.float32),
            pltpu.VMEM((_BPW + _LANES,), jnp.float32),
            pltpu.SemaphoreType.DMA,
            pltpu.SemaphoreType.DMA,
        ],
    )
    def dot_gather(id1_hbm, id2_hbm, table_hbm, out_hbm,
                   idx1_v, idx2_v, rows1_v, rows2_v, a_v, o_v, sem1, sem2):
        wid = lax.axis_index("s") * _NC + lax.axis_index("c")
        base = wid * _BPW
        pltpu.sync_copy(id1_hbm.at[pl.ds(base, _BPW)], idx1_v)
        pltpu.sync_copy(id2_hbm.at[pl.ds(base, _BPW)], idx2_v)
        c1 = pltpu.async_copy(table_hbm.at[idx1_v], rows1_v, sem1)
        c2 = pltpu.async_copy(table_hbm.at[idx2_v], rows2_v, sem2)
        c1.wait()
        c2.wait()

        lanes = lax.iota(jnp.int32, _LANES)
        lane0 = lanes == 0

        def group(t, carry):
            for i in range(_LANES):
                r = t * _LANES + i
                c = r * _LANES
                s = (rows1_v[r, pl.ds(0, _LANES)]
                     * rows2_v[r, pl.ds(0, _LANES)]
                     + rows1_v[r, pl.ds(_LANES, _LANES)]
                     * rows2_v[r, pl.ds(_LANES, _LANES)])
                a_v[pl.ds(c, _LANES)] = s
                v = s + a_v[pl.ds(c + 8, _LANES)]
                a_v[pl.ds(c, _LANES)] = v
                v = v + a_v[pl.ds(c + 4, _LANES)]
                a_v[pl.ds(c, _LANES)] = v
                v = v + a_v[pl.ds(c + 2, _LANES)]
                a_v[pl.ds(c, _LANES)] = v
                v = v + a_v[pl.ds(c + 1, _LANES)]
                plsc.store_compressed(o_v.at[pl.ds(r, _LANES)], v, mask=lane0)
            return carry

        lax.fori_loop(0, _GROUPS, group, 0)
        pltpu.sync_copy(o_v.at[pl.ds(0, _BPW)], out_hbm.at[pl.ds(base, _BPW)])

    return dot_gather


_sc_kernel = _make_sc_kernel()


def kernel(loc_id1, loc_id2, V_loc):
    return _sc_kernel(loc_id1, loc_id2, V_loc)
